# baseline (speedup 1.0000x reference)
import functools

import jax
import jax.numpy as jnp
from jax import lax
from jax.experimental import pallas as pl
from jax.experimental.pallas import tpu as pltpu
from jax.experimental.pallas import tpu_sc as plsc

_CHUNK = 128
_BCH = 512
_LANES = 16
_PITCH = 17


def _make_gather(vocab: int, emb: int, hist: int, batch: int):
    info = plsc.get_sparse_core_info()
    nw = info.num_cores * info.num_subcores
    assert emb % _LANES == 0 and batch % _BCH == 0
    egrp = emb // 8
    bgrp = _BCH // _CHUNK
    eblk = emb // _LANES
    ntiles = hist * (batch // _BCH)
    assert ntiles % nw == 0
    tiles_per_w = ntiles // nw
    rows_per_w = tiles_per_w * _BCH
    tpb = batch // _BCH

    mesh = plsc.VectorSubcoreMesh(core_axis_name="c", subcore_axis_name="s")

    @functools.partial(
        pl.kernel,
        mesh=mesh,
        out_type=jax.ShapeDtypeStruct(
            (hist, egrp, batch // _CHUNK, 8, _CHUNK), jnp.float32),
        scratch_types=[
            pltpu.VMEM((rows_per_w,), jnp.int32),
            pltpu.VMEM((_CHUNK, emb), jnp.float32),
            pltpu.VMEM((_CHUNK, emb), jnp.float32),
            pltpu.VMEM((_CHUNK, _PITCH), jnp.float32),
            pltpu.VMEM((egrp, bgrp, 8, _CHUNK), jnp.float32),
            pltpu.VMEM((egrp, bgrp, 8, _CHUNK), jnp.float32),
            pltpu.SemaphoreType.DMA,
            pltpu.SemaphoreType.DMA,
            pltpu.SemaphoreType.DMA,
            pltpu.SemaphoreType.DMA,
        ],
        compiler_params=pltpu.CompilerParams(
            use_tc_tiling_on_sc=False, needs_layout_passes=False),
    )
    def gather_kernel(table_hbm, idx_hbm, out_hbm, idx_v, g_buf0, g_buf1,
                      skew, t_buf0, t_buf1, gsem0, gsem1, osem0, osem1):
        g_bufs = (g_buf0, g_buf1)
        t_bufs = (t_buf0, t_buf1)
        gsems = (gsem0, gsem1)
        osems = (osem0, osem1)
        wid = lax.axis_index("s") * info.num_cores + lax.axis_index("c")
        base = wid * rows_per_w
        pltpu.sync_copy(idx_hbm.at[pl.ds(base, rows_per_w)], idx_v)

        iota = lax.iota(jnp.int32, _LANES)
        rowsel = [iota + _LANES * j for j in range(_CHUNK // _LANES)]

        def gather_desc(i_local, c, gb):
            off = i_local * _BCH + c * _CHUNK
            return pltpu.make_async_copy(
                table_hbm.at[idx_v.at[pl.ds(off, _CHUNK)]],
                g_bufs[gb],
                gsems[gb],
            )

        def out_descs(tb, h, tau):
            return [
                pltpu.make_async_copy(
                    t_bufs[tb].at[g],
                    out_hbm.at[h, g, pl.ds(tau * bgrp, bgrp)],
                    osems[tb],
                )
                for g in range(egrp)
            ]

        def transpose_chunk(gb, tb, c):
            def colblk(kk):
                def p1(r0):
                    for dr in range(8):
                        rr = r0 + dr
                        skew[rr, pl.ds(0, _LANES)] = (
                            g_bufs[gb][rr, pl.ds(kk * _LANES, _LANES)])

                plsc.parallel_loop(0, _CHUNK, 8, unroll=8)(p1)

                def p2(el):
                    e = kk * _LANES + el
                    g = e // 8
                    e8 = lax.rem(e, 8)
                    evec = jnp.full((_LANES,), el, jnp.int32)
                    for j in range(_CHUNK // _LANES):
                        v = plsc.load_gather(skew, [rowsel[j], evec])
                        t_bufs[tb][g, c, e8, pl.ds(j * _LANES, _LANES)] = v

                plsc.parallel_loop(0, _LANES, unroll=8)(p2)

            pl.loop(0, eblk)(colblk)

        def pair(p):
            for tb in range(2):
                i_local = 2 * p + tb
                t = wid * tiles_per_w + i_local
                h = t // tpb
                tau = lax.rem(t, tpb)

                @pl.when(p >= 1)
                def _():
                    for d in out_descs(tb, h, tau):
                        d.wait()

                gather_desc(i_local, 0, 0).start()
                gather_desc(i_local, 1, 1).start()
                for c in range(bgrp):
                    gb = c % 2
                    gather_desc(i_local, c, gb).wait()
                    transpose_chunk(gb, tb, c)
                    if c + 2 < bgrp:
                        gather_desc(i_local, c + 2, gb).start()
                for d in out_descs(tb, h, tau):
                    d.start()

        pl.loop(0, tiles_per_w // 2)(pair)

        for tb in range(2):
            for d in out_descs(tb, 0, 0):
                d.wait()

    return gather_kernel


def kernel(indices, table):
    batch, hist = indices.shape
    vocab, emb = table.shape
    idx_flat = indices.T.reshape(-1)
    out = _make_gather(vocab, emb, hist, batch)(table, idx_flat)
    out = out.transpose(0, 2, 4, 1, 3).reshape(hist, batch, emb)
    return out


# speedup vs baseline: 1.4819x; 1.4819x over previous
import functools

import jax
import jax.numpy as jnp
from jax import lax
from jax.experimental import pallas as pl
from jax.experimental.pallas import tpu as pltpu
from jax.experimental.pallas import tpu_sc as plsc

_CHUNK = 128
_BCH = 512
_LANES = 16
_PITCH = 17


def _make_gather(vocab: int, emb: int, hist: int, batch: int):
    info = plsc.get_sparse_core_info()
    nw = info.num_cores * info.num_subcores
    assert emb % _LANES == 0 and batch % _BCH == 0
    egrp = emb // 8
    bgrp = _BCH // _CHUNK
    eblk = emb // _LANES
    ntiles = hist * (batch // _BCH)
    assert ntiles % nw == 0
    tiles_per_w = ntiles // nw
    rows_per_w = tiles_per_w * _BCH
    tpb = batch // _BCH

    mesh = plsc.VectorSubcoreMesh(core_axis_name="c", subcore_axis_name="s")

    @functools.partial(
        pl.kernel,
        mesh=mesh,
        out_type=jax.ShapeDtypeStruct(
            (hist, egrp, batch // _CHUNK, 8, _CHUNK), jnp.float32),
        scratch_types=[
            pltpu.VMEM((rows_per_w,), jnp.int32),
            pltpu.VMEM((_CHUNK, emb), jnp.float32),
            pltpu.VMEM((_CHUNK, emb), jnp.float32),
            pltpu.VMEM((_CHUNK, _PITCH), jnp.float32),
            pltpu.VMEM((egrp, bgrp, 8, _CHUNK), jnp.float32),
            pltpu.VMEM((egrp, bgrp, 8, _CHUNK), jnp.float32),
            pltpu.SemaphoreType.DMA,
            pltpu.SemaphoreType.DMA,
            pltpu.SemaphoreType.DMA,
            pltpu.SemaphoreType.DMA,
        ],
        compiler_params=pltpu.CompilerParams(
            use_tc_tiling_on_sc=False, needs_layout_passes=False),
    )
    def gather_kernel(table_hbm, idx_hbm, out_hbm, idx_v, g_buf0, g_buf1,
                      skew, t_buf0, t_buf1, gsem0, gsem1, osem0, osem1):
        g_bufs = (g_buf0, g_buf1)
        t_bufs = (t_buf0, t_buf1)
        gsems = (gsem0, gsem1)
        osems = (osem0, osem1)
        wid = lax.axis_index("s") * info.num_cores + lax.axis_index("c")
        base = wid * rows_per_w
        pltpu.sync_copy(idx_hbm.at[pl.ds(base, rows_per_w)], idx_v)

        iota = lax.iota(jnp.int32, _LANES)
        rowsel = [iota + _LANES * j for j in range(_CHUNK // _LANES)]

        def gather_desc(i_local, c, gb):
            off = i_local * _BCH + c * _CHUNK
            return pltpu.make_async_copy(
                table_hbm.at[idx_v.at[pl.ds(off, _CHUNK)]],
                g_bufs[gb],
                gsems[gb],
            )

        def out_descs(tb, h, tau):
            return [
                pltpu.make_async_copy(
                    t_bufs[tb].at[g],
                    out_hbm.at[h, g, pl.ds(tau * bgrp, bgrp)],
                    osems[tb],
                )
                for g in range(egrp)
            ]

        def transpose_chunk(gb, tb, c):
            def colblk(kk):
                def p1(r0):
                    for dr in range(8):
                        rr = r0 + dr
                        skew[rr, pl.ds(0, _LANES)] = (
                            g_bufs[gb][rr, pl.ds(kk * _LANES, _LANES)])

                plsc.parallel_loop(0, _CHUNK, 8, unroll=4)(p1)

                def p2(el):
                    e = kk * _LANES + el
                    g = e // 8
                    e8 = lax.rem(e, 8)
                    evec = jnp.full((_LANES,), el, jnp.int32)
                    for j in range(_CHUNK // _LANES):
                        v = plsc.load_gather(skew, [rowsel[j], evec])
                        t_bufs[tb][g, c, e8, pl.ds(j * _LANES, _LANES)] = v

                plsc.parallel_loop(0, _LANES, unroll=4)(p2)

            pl.loop(0, eblk)(colblk)

        npairs = tiles_per_w // 2

        def pair(p):
            for tb in range(2):
                i_local = 2 * p + tb
                t = wid * tiles_per_w + i_local
                h = t // tpb
                tau = lax.rem(t, tpb)

                @pl.when(p >= 1)
                def _():
                    for d in out_descs(tb, h, tau):
                        d.wait()

                for c in range(bgrp):
                    gb = c % 2
                    gather_desc(i_local, c, gb).wait()
                    transpose_chunk(gb, tb, c)
                    if c + 2 < bgrp:
                        gather_desc(i_local, c + 2, gb).start()
                    elif tb == 0:
                        gather_desc(i_local + 1, c + 2 - bgrp, gb).start()
                    else:
                        @pl.when(p + 1 < npairs)
                        def _():
                            gather_desc(i_local + 1, c + 2 - bgrp, gb).start()
                for d in out_descs(tb, h, tau):
                    d.start()

        gather_desc(0, 0, 0).start()
        gather_desc(0, 1, 1).start()
        pl.loop(0, npairs)(pair)

        for tb in range(2):
            for d in out_descs(tb, 0, 0):
                d.wait()

    return gather_kernel


def kernel(indices, table):
    batch, hist = indices.shape
    vocab, emb = table.shape
    idx_flat = indices.T.reshape(-1)
    out = _make_gather(vocab, emb, hist, batch)(table, idx_flat)
    out = out.transpose(0, 2, 4, 1, 3).reshape(hist, batch, emb)
    return out


# speedup vs baseline: 1.5288x; 1.0317x over previous
import functools

import jax
import jax.numpy as jnp
from jax import lax
from jax.experimental import pallas as pl
from jax.experimental.pallas import tpu as pltpu
from jax.experimental.pallas import tpu_sc as plsc

_CHUNK = 128
_BCH = 512
_LANES = 16
_PITCH = 17


def _make_gather(vocab: int, emb: int, hist: int, batch: int):
    info = plsc.get_sparse_core_info()
    nw = info.num_cores * info.num_subcores
    assert emb % _LANES == 0 and batch % _BCH == 0
    egrp = emb // 8
    bgrp = _BCH // _CHUNK
    eblk = emb // _LANES
    ntiles = hist * (batch // _BCH)
    assert ntiles % nw == 0
    tiles_per_w = ntiles // nw
    rows_per_w = tiles_per_w * _BCH
    tpb = batch // _BCH

    mesh = plsc.VectorSubcoreMesh(core_axis_name="c", subcore_axis_name="s")

    @functools.partial(
        pl.kernel,
        mesh=mesh,
        out_type=jax.ShapeDtypeStruct(
            (hist, egrp, batch // _CHUNK, 8, _CHUNK), jnp.float32),
        scratch_types=[
            pltpu.VMEM((rows_per_w,), jnp.int32),
            pltpu.VMEM((_CHUNK, emb), jnp.float32),
            pltpu.VMEM((_CHUNK, emb), jnp.float32),
            pltpu.VMEM((_CHUNK * _PITCH,), jnp.float32),
            pltpu.VMEM((egrp, bgrp, 8, _CHUNK), jnp.float32),
            pltpu.VMEM((egrp, bgrp, 8, _CHUNK), jnp.float32),
            pltpu.SemaphoreType.DMA,
            pltpu.SemaphoreType.DMA,
            pltpu.SemaphoreType.DMA,
            pltpu.SemaphoreType.DMA,
        ],
        compiler_params=pltpu.CompilerParams(
            use_tc_tiling_on_sc=False, needs_layout_passes=False),
    )
    def gather_kernel(table_hbm, idx_hbm, out_hbm, idx_v, g_buf0, g_buf1,
                      skew, t_buf0, t_buf1, gsem0, gsem1, osem0, osem1):
        g_bufs = (g_buf0, g_buf1)
        t_bufs = (t_buf0, t_buf1)
        gsems = (gsem0, gsem1)
        osems = (osem0, osem1)
        wid = lax.axis_index("s") * info.num_cores + lax.axis_index("c")
        base = wid * rows_per_w
        pltpu.sync_copy(idx_hbm.at[pl.ds(base, rows_per_w)], idx_v)

        iota = lax.iota(jnp.int32, _LANES)
        rowsel17 = [(iota + _LANES * j) * _PITCH
                    for j in range(_CHUNK // _LANES)]

        def gather_desc(i_local, c, gb):
            off = i_local * _BCH + c * _CHUNK
            return pltpu.make_async_copy(
                table_hbm.at[idx_v.at[pl.ds(off, _CHUNK)]],
                g_bufs[gb],
                gsems[gb],
            )

        def out_descs(tb, h, tau):
            return [
                pltpu.make_async_copy(
                    t_bufs[tb].at[g],
                    out_hbm.at[h, g, pl.ds(tau * bgrp, bgrp)],
                    osems[tb],
                )
                for g in range(egrp)
            ]

        def transpose_chunk(gb, tb, c):
            def colblk(kk):
                def p1(r0):
                    for dr in range(8):
                        rr = r0 + dr
                        skew[pl.ds(rr * _PITCH, _LANES)] = (
                            g_bufs[gb][rr, pl.ds(kk * _LANES, _LANES)])

                plsc.parallel_loop(0, _CHUNK, 8, unroll=4)(p1)

                def p2(el):
                    e = kk * _LANES + el
                    g = e // 8
                    e8 = lax.rem(e, 8)
                    evec = jnp.full((_LANES,), el, jnp.int32)
                    for j in range(_CHUNK // _LANES):
                        v = plsc.load_gather(skew, [rowsel17[j] + evec])
                        t_bufs[tb][g, c, e8, pl.ds(j * _LANES, _LANES)] = v

                plsc.parallel_loop(0, _LANES, unroll=4)(p2)

            pl.loop(0, eblk)(colblk)

        npairs = tiles_per_w // 2

        def pair(p):
            for tb in range(2):
                i_local = 2 * p + tb
                t = wid * tiles_per_w + i_local
                h = t // tpb
                tau = lax.rem(t, tpb)

                @pl.when(p >= 1)
                def _():
                    for d in out_descs(tb, h, tau):
                        d.wait()

                for c in range(bgrp):
                    gb = c % 2
                    gather_desc(i_local, c, gb).wait()
                    transpose_chunk(gb, tb, c)
                    if c + 2 < bgrp:
                        gather_desc(i_local, c + 2, gb).start()
                    elif tb == 0:
                        gather_desc(i_local + 1, c + 2 - bgrp, gb).start()
                    else:
                        @pl.when(p + 1 < npairs)
                        def _():
                            gather_desc(i_local + 1, c + 2 - bgrp, gb).start()
                for d in out_descs(tb, h, tau):
                    d.start()

        gather_desc(0, 0, 0).start()
        gather_desc(0, 1, 1).start()
        pl.loop(0, npairs)(pair)

        for tb in range(2):
            for d in out_descs(tb, 0, 0):
                d.wait()

    return gather_kernel


def kernel(indices, table):
    batch, hist = indices.shape
    vocab, emb = table.shape
    idx_flat = indices.T.reshape(-1)
    out = _make_gather(vocab, emb, hist, batch)(table, idx_flat)
    out = out.transpose(0, 2, 4, 1, 3).reshape(hist, batch, emb)
    return out


# speedup vs baseline: 1.7641x; 1.1539x over previous
import functools

import jax
import jax.numpy as jnp
from jax import lax
from jax.experimental import pallas as pl
from jax.experimental.pallas import tpu as pltpu
from jax.experimental.pallas import tpu_sc as plsc

_CHUNK = 128
_BCH = 512
_LANES = 16
_PITCH = 17


def _make_gather(vocab: int, emb: int, hist: int, batch: int):
    info = plsc.get_sparse_core_info()
    nw = info.num_cores * info.num_subcores
    assert emb % _LANES == 0 and batch % _BCH == 0
    egrp = emb // 8
    bgrp = _BCH // _CHUNK
    eblk = emb // _LANES
    ntiles = hist * (batch // _BCH)
    assert ntiles % nw == 0
    tiles_per_w = ntiles // nw
    rows_per_w = tiles_per_w * _BCH
    tpb = batch // _BCH

    mesh = plsc.VectorSubcoreMesh(core_axis_name="c", subcore_axis_name="s")

    @functools.partial(
        pl.kernel,
        mesh=mesh,
        out_type=jax.ShapeDtypeStruct(
            (hist, egrp, batch // _CHUNK, 8, _CHUNK), jnp.float32),
        scratch_types=[
            pltpu.VMEM((rows_per_w,), jnp.int32),
            pltpu.VMEM((_CHUNK, emb), jnp.float32),
            pltpu.VMEM((_CHUNK, emb), jnp.float32),
            pltpu.VMEM((_CHUNK, emb), jnp.float32),
            pltpu.VMEM((_CHUNK, emb), jnp.float32),
            pltpu.VMEM((_CHUNK * _PITCH,), jnp.float32),
            pltpu.VMEM((egrp, bgrp, 8, _CHUNK), jnp.float32),
            pltpu.VMEM((egrp, bgrp, 8, _CHUNK), jnp.float32),
            pltpu.SemaphoreType.DMA,
            pltpu.SemaphoreType.DMA,
            pltpu.SemaphoreType.DMA,
            pltpu.SemaphoreType.DMA,
            pltpu.SemaphoreType.DMA,
            pltpu.SemaphoreType.DMA,
        ],
        compiler_params=pltpu.CompilerParams(
            use_tc_tiling_on_sc=False, needs_layout_passes=False),
    )
    def gather_kernel(table_hbm, idx_hbm, out_hbm, idx_v, g_buf0, g_buf1,
                      g_buf2, g_buf3, skew, t_buf0, t_buf1,
                      gsem0, gsem1, gsem2, gsem3, osem0, osem1):
        g_bufs = (g_buf0, g_buf1, g_buf2, g_buf3)
        t_bufs = (t_buf0, t_buf1)
        gsems = (gsem0, gsem1, gsem2, gsem3)
        osems = (osem0, osem1)
        wid = lax.axis_index("s") * info.num_cores + lax.axis_index("c")
        base = wid * rows_per_w
        pltpu.sync_copy(idx_hbm.at[pl.ds(base, rows_per_w)], idx_v)

        iota = lax.iota(jnp.int32, _LANES)
        rowsel17 = [(iota + _LANES * j) * _PITCH
                    for j in range(_CHUNK // _LANES)]

        def gather_desc(i_local, c, gb):
            off = i_local * _BCH + c * _CHUNK
            return pltpu.make_async_copy(
                table_hbm.at[idx_v.at[pl.ds(off, _CHUNK)]],
                g_bufs[gb],
                gsems[gb],
            )

        def out_descs(tb, h, tau):
            return [
                pltpu.make_async_copy(
                    t_bufs[tb].at[g],
                    out_hbm.at[h, g, pl.ds(tau * bgrp, bgrp)],
                    osems[tb],
                )
                for g in range(egrp)
            ]

        def transpose_chunk(gb, tb, c):
            def colblk(kk):
                def p1(r0):
                    for dr in range(8):
                        rr = r0 + dr
                        skew[pl.ds(rr * _PITCH, _LANES)] = (
                            g_bufs[gb][rr, pl.ds(kk * _LANES, _LANES)])

                plsc.parallel_loop(0, _CHUNK, 8, unroll=4)(p1)

                def p2(el):
                    e = kk * _LANES + el
                    g = e // 8
                    e8 = lax.rem(e, 8)
                    evec = jnp.full((_LANES,), el, jnp.int32)
                    for j in range(_CHUNK // _LANES):
                        v = plsc.load_gather(skew, [rowsel17[j] + evec])
                        t_bufs[tb][g, c, e8, pl.ds(j * _LANES, _LANES)] = v

                plsc.parallel_loop(0, _LANES, unroll=4)(p2)

            pl.loop(0, eblk)(colblk)

        npairs = tiles_per_w // 2

        def pair(p):
            for tb in range(2):
                i_local = 2 * p + tb
                t = wid * tiles_per_w + i_local
                h = t // tpb
                tau = lax.rem(t, tpb)

                @pl.when(p >= 1)
                def _():
                    for d in out_descs(tb, h, tau):
                        d.wait()

                for c in range(bgrp):
                    gb = c
                    gather_desc(i_local, c, gb).wait()
                    transpose_chunk(gb, tb, c)
                    if tb == 0:
                        gather_desc(i_local + 1, c, gb).start()
                    else:
                        @pl.when(p + 1 < npairs)
                        def _():
                            gather_desc(i_local + 1, c, gb).start()
                for d in out_descs(tb, h, tau):
                    d.start()

        for c0 in range(bgrp):
            gather_desc(0, c0, c0).start()
        pl.loop(0, npairs)(pair)

        for tb in range(2):
            for d in out_descs(tb, 0, 0):
                d.wait()

    return gather_kernel


def kernel(indices, table):
    batch, hist = indices.shape
    vocab, emb = table.shape
    idx_flat = indices.T.reshape(-1)
    out = _make_gather(vocab, emb, hist, batch)(table, idx_flat)
    out = out.transpose(0, 2, 4, 1, 3).reshape(hist, batch, emb)
    return out
